# Initial kernel scaffold; baseline (speedup 1.0000x reference)
import jax
import jax.numpy as jnp
from jax.experimental import pallas as pl


def kernel(seq_len, pos_embedding):
    raise NotImplementedError("write your pallas kernel here")


# speedup vs baseline: 1.4072x; 1.4072x over previous
import functools

import jax
import jax.numpy as jnp
from jax import lax
from jax.experimental import pallas as pl
from jax.experimental.pallas import tpu as pltpu
from jax.experimental.pallas import tpu_sc as plsc

SEQ_LEN = 4096
EMB = 1024
NUM_CORES = 2
NUM_SUBCORES = 16
NUM_WORKERS = NUM_CORES * NUM_SUBCORES
ROWS_PER_WORKER = SEQ_LEN // NUM_WORKERS
CHUNK = 32
NUM_CHUNKS = ROWS_PER_WORKER // CHUNK

@functools.lru_cache(maxsize=1)
def _build_copy_rows():
    mesh = plsc.VectorSubcoreMesh(
        core_axis_name="c", subcore_axis_name="s",
        num_cores=NUM_CORES, num_subcores=NUM_SUBCORES)

    @functools.partial(
        pl.kernel,
        out_type=jax.ShapeDtypeStruct((SEQ_LEN, EMB), jnp.float32),
        mesh=mesh,
        scratch_types=[
            pltpu.VMEM((CHUNK, EMB), jnp.float32),
            pltpu.VMEM((CHUNK, EMB), jnp.float32),
            pltpu.SemaphoreType.DMA,
            pltpu.SemaphoreType.DMA,
            pltpu.SemaphoreType.DMA,
            pltpu.SemaphoreType.DMA,
        ],
    )
    def copy_rows(table_hbm, out_hbm, buf0, buf1, isem0, isem1, osem0, osem1):
        wid = lax.axis_index("s") * NUM_CORES + lax.axis_index("c")
        base = wid * ROWS_PER_WORKER
        bufs = (buf0, buf1)
        isems = (isem0, isem1)
        osems = (osem0, osem1)

        def in_copy(i, b):
            return pltpu.make_async_copy(
                table_hbm.at[pl.ds(base + i * CHUNK, CHUNK)], bufs[b], isems[b])

        def out_copy(i, b):
            return pltpu.make_async_copy(
                bufs[b], out_hbm.at[pl.ds(base + i * CHUNK, CHUNK)], osems[b])

        in_copy(0, 0).start()
        for i in range(NUM_CHUNKS):
            cur = i % 2
            nxt = (i + 1) % 2
            in_copy(i, cur).wait()
            out_copy(i, cur).start()
            if i + 1 < NUM_CHUNKS:
                if i >= 1:
                    out_copy(i - 1, nxt).wait()
                in_copy(i + 1, nxt).start()
        out_copy(NUM_CHUNKS - 2, NUM_CHUNKS % 2).wait()
        out_copy(NUM_CHUNKS - 1, (NUM_CHUNKS - 1) % 2).wait()

    return copy_rows


def kernel(seq_len, pos_embedding):
    del seq_len
    return _build_copy_rows()(pos_embedding)
